# baseline (speedup 1.0000x reference)
import functools

import jax
import jax.numpy as jnp
import numpy as np
from jax import lax
from jax.experimental import pallas as pl
from jax.experimental.pallas import tpu as pltpu
from jax.experimental.pallas import tpu_sc as plsc

N = 10000
E = 320000
F = 128
H = 128

NC = 2
NS = 16
NW = NC * NS
EPT = E // NW
C = 80
NCH = EPT // C
NBUF = 4
LOOKAHEAD = 2
SINFLIGHT = 2
NOUT = NCH // NBUF
NREM = NCH - NOUT * NBUF

RPT = 632
RPT_LAST = N - (NS - 1) * RPT

_LOG2 = float(np.log(2.0))


def _sc_segment_sum(e, idx, zrows):
    mesh = plsc.VectorSubcoreMesh(core_axis_name="c", subcore_axis_name="s")

    @functools.partial(
        pl.kernel,
        mesh=mesh,
        out_type=jax.ShapeDtypeStruct((NC, N, F), jnp.float32),
        scratch_types=[
            pltpu.VMEM_SHARED((N, F), jnp.float32),
            [pltpu.VMEM((C, F), jnp.float32) for _ in range(NBUF)],
            [pltpu.VMEM((C,), jnp.int32) for _ in range(NBUF)],
            [pltpu.SemaphoreType.DMA for _ in range(NBUF)],
            [pltpu.SemaphoreType.DMA for _ in range(NBUF)],
        ],
    )
    def seg_sum(e_hbm, idx_hbm, z_hbm, out_hbm, acc, ebufs, ibufs, lsems, ssems):
        c = lax.axis_index("c")
        s = lax.axis_index("s")
        wid = s * NC + c
        ebase = wid * EPT

        off = pl.multiple_of(s * RPT, 8)

        @pl.when(s < NS - 1)
        def _():
            pltpu.sync_copy(z_hbm, acc.at[pl.ds(off, RPT)])

        @pl.when(s == NS - 1)
        def _():
            pltpu.sync_copy(
                z_hbm.at[pl.ds(0, RPT_LAST)], acc.at[pl.ds(off, RPT_LAST)]
            )

        def fire_load(j, b):
            pltpu.async_copy(
                e_hbm.at[pl.ds(ebase + j * C, C)], ebufs[b], lsems[b]
            )
            pltpu.async_copy(
                idx_hbm.at[pl.ds(ebase + j * C, C)], ibufs[b], lsems[b]
            )

        def wait_load(j, b):
            pltpu.make_async_copy(
                e_hbm.at[pl.ds(ebase + j * C, C)], ebufs[b], lsems[b]
            ).wait()
            pltpu.make_async_copy(
                idx_hbm.at[pl.ds(ebase + j * C, C)], ibufs[b], lsems[b]
            ).wait()

        def wait_scatter(b):
            pltpu.make_async_copy(ebufs[b], acc.at[ibufs[b]], ssems[b]).wait()

        for b in range(LOOKAHEAD):
            fire_load(b, b)

        plsc.subcore_barrier()

        def process_chunk(j, b, guard_drain, may_prefetch):
            wait_load(j, b)
            pltpu.async_copy(ebufs[b], acc.at[ibufs[b]], ssems[b], add=True)
            bn = (b + SINFLIGHT) % NBUF
            if guard_drain:
                @pl.when(j >= SINFLIGHT)
                def _():
                    wait_scatter(bn)
            else:
                wait_scatter(bn)
            if may_prefetch:
                @pl.when(j + LOOKAHEAD < NCH)
                def _():
                    fire_load(j + LOOKAHEAD, (b + LOOKAHEAD) % NBUF)

        def body(g, carry):
            for b in range(NBUF):
                process_chunk(g * NBUF + b, b, True, True)
            return carry

        lax.fori_loop(0, NOUT, body, 0)
        for r in range(NREM):
            j = NOUT * NBUF + r
            process_chunk(j, j % NBUF, False, j + LOOKAHEAD < NCH)

        for j in range(NCH - SINFLIGHT, NCH):
            wait_scatter(j % NBUF)

        plsc.subcore_barrier()

        @pl.when(s < NS - 1)
        def _():
            pltpu.sync_copy(
                acc.at[pl.ds(off, RPT)], out_hbm.at[c, pl.ds(off, RPT)]
            )

        @pl.when(s == NS - 1)
        def _():
            pltpu.sync_copy(
                acc.at[pl.ds(off, RPT_LAST)],
                out_hbm.at[c, pl.ds(off, RPT_LAST)],
            )

    return seg_sum(e, idx, zrows)


def _mlp(p, v, w1, b1r, w2, b2r):
    bn = 1000

    def body(p_ref, v_ref, w1_ref, b1_ref, w2_ref, b2_ref, o_ref):
        ssum = p_ref[0] + p_ref[1]
        h = lax.dot_general(
            ssum, w1_ref[...], (((1,), (1,)), ((), ())),
            preferred_element_type=jnp.float32,
        )
        h = h + b1_ref[...]
        sp = jnp.maximum(h, 0.0) + jnp.log1p(jnp.exp(-jnp.abs(h))) - _LOG2
        o = lax.dot_general(
            sp, w2_ref[...], (((1,), (1,)), ((), ())),
            preferred_element_type=jnp.float32,
        )
        o_ref[...] = o + b2_ref[...] + v_ref[...]

    return pl.pallas_call(
        body,
        grid=(N // bn,),
        in_specs=[
            pl.BlockSpec((NC, bn, H), lambda i: (0, i, 0)),
            pl.BlockSpec((bn, H), lambda i: (i, 0)),
            pl.BlockSpec((H, F), lambda i: (0, 0)),
            pl.BlockSpec((1, H), lambda i: (0, 0)),
            pl.BlockSpec((H, H), lambda i: (0, 0)),
            pl.BlockSpec((1, H), lambda i: (0, 0)),
        ],
        out_specs=pl.BlockSpec((bn, H), lambda i: (i, 0)),
        out_shape=jax.ShapeDtypeStruct((N, H), jnp.float32),
    )(p, v, w1, b1r, w2, b2r)


def kernel(v, e, edge_index, W1, b1, W2, b2):
    idx = edge_index[1]
    zrows = jnp.zeros((RPT, F), jnp.float32)
    partials = _sc_segment_sum(e, idx, zrows)
    return _mlp(
        partials,
        v,
        W1,
        b1.reshape(1, H),
        W2,
        b2.reshape(1, H),
    )


# speedup vs baseline: 1.1166x; 1.1166x over previous
import functools

import jax
import jax.numpy as jnp
import numpy as np
from jax import lax
from jax.experimental import pallas as pl
from jax.experimental.pallas import tpu as pltpu
from jax.experimental.pallas import tpu_sc as plsc

N = 10000
E = 320000
F = 128
H = 128

NC = 2
NS = 16
NW = NC * NS
EPT = E // NW
C = 80
NCH = EPT // C
NBUF = 3
LOOKAHEAD = 2
NOUT = NCH // NBUF
NREM = NCH - NOUT * NBUF

RPT = 632
RPT_LAST = N - (NS - 1) * RPT

_LOG2 = float(np.log(2.0))


def _sc_segment_sum(e, idx, zrows):
    mesh = plsc.VectorSubcoreMesh(core_axis_name="c", subcore_axis_name="s")

    @functools.partial(
        pl.kernel,
        mesh=mesh,
        out_type=jax.ShapeDtypeStruct((NC, N, F), jnp.float32),
        scratch_types=[
            pltpu.VMEM_SHARED((N, F), jnp.float32),
            pltpu.VMEM((NCH, C), jnp.int32),
            [pltpu.VMEM((C, F), jnp.float32) for _ in range(NBUF)],
            [pltpu.SemaphoreType.DMA for _ in range(NBUF)],
        ],
    )
    def seg_sum(e_hbm, idx_hbm, z_hbm, out_hbm, acc, ibuf, ebufs, lsems):
        c = lax.axis_index("c")
        s = lax.axis_index("s")
        wid = s * NC + c
        ebase = wid * EPT

        pltpu.sync_copy(idx_hbm.at[wid], ibuf)

        off = pl.multiple_of(s * RPT, 8)

        @pl.when(s < NS - 1)
        def _():
            pltpu.sync_copy(z_hbm, acc.at[pl.ds(off, RPT)])

        @pl.when(s == NS - 1)
        def _():
            pltpu.sync_copy(
                z_hbm.at[pl.ds(0, RPT_LAST)], acc.at[pl.ds(off, RPT_LAST)]
            )

        def fire_load(j, b):
            pltpu.async_copy(
                e_hbm.at[pl.ds(ebase + j * C, C)], ebufs[b], lsems[b]
            )

        def wait_load(j, b):
            pltpu.make_async_copy(
                e_hbm.at[pl.ds(ebase + j * C, C)], ebufs[b], lsems[b]
            ).wait()

        for b in range(LOOKAHEAD):
            fire_load(b, b)

        plsc.subcore_barrier()

        def process_chunk(j, b, may_prefetch):
            wait_load(j, b)
            if may_prefetch:
                @pl.when(j + LOOKAHEAD < NCH)
                def _():
                    fire_load(j + LOOKAHEAD, (b + LOOKAHEAD) % NBUF)
            pltpu.sync_copy(ebufs[b], acc.at[ibuf.at[j]], add=True)

        def body(g, carry):
            for b in range(NBUF):
                process_chunk(g * NBUF + b, b, True)
            return carry

        lax.fori_loop(0, NOUT, body, 0)
        for r in range(NREM):
            j = NOUT * NBUF + r
            process_chunk(j, j % NBUF, j + LOOKAHEAD < NCH)

        plsc.subcore_barrier()

        @pl.when(s < NS - 1)
        def _():
            pltpu.sync_copy(
                acc.at[pl.ds(off, RPT)], out_hbm.at[c, pl.ds(off, RPT)]
            )

        @pl.when(s == NS - 1)
        def _():
            pltpu.sync_copy(
                acc.at[pl.ds(off, RPT_LAST)],
                out_hbm.at[c, pl.ds(off, RPT_LAST)],
            )

    return seg_sum(e, idx, zrows)


def _mlp(p, v, w1, b1r, w2, b2r):
    bn = 1000

    def body(p_ref, v_ref, w1_ref, b1_ref, w2_ref, b2_ref, o_ref):
        ssum = p_ref[0] + p_ref[1]
        h = lax.dot_general(
            ssum, w1_ref[...], (((1,), (1,)), ((), ())),
            preferred_element_type=jnp.float32,
        )
        h = h + b1_ref[...]
        sp = jnp.maximum(h, 0.0) + jnp.log1p(jnp.exp(-jnp.abs(h))) - _LOG2
        o = lax.dot_general(
            sp, w2_ref[...], (((1,), (1,)), ((), ())),
            preferred_element_type=jnp.float32,
        )
        o_ref[...] = o + b2_ref[...] + v_ref[...]

    return pl.pallas_call(
        body,
        grid=(N // bn,),
        in_specs=[
            pl.BlockSpec((NC, bn, H), lambda i: (0, i, 0)),
            pl.BlockSpec((bn, H), lambda i: (i, 0)),
            pl.BlockSpec((H, F), lambda i: (0, 0)),
            pl.BlockSpec((1, H), lambda i: (0, 0)),
            pl.BlockSpec((H, H), lambda i: (0, 0)),
            pl.BlockSpec((1, H), lambda i: (0, 0)),
        ],
        out_specs=pl.BlockSpec((bn, H), lambda i: (i, 0)),
        out_shape=jax.ShapeDtypeStruct((N, H), jnp.float32),
    )(p, v, w1, b1r, w2, b2r)


def kernel(v, e, edge_index, W1, b1, W2, b2):
    idx = edge_index[1].reshape(NW, NCH, C)
    zrows = jnp.zeros((RPT, F), jnp.float32)
    partials = _sc_segment_sum(e, idx, zrows)
    return _mlp(
        partials,
        v,
        W1,
        b1.reshape(1, H),
        W2,
        b2.reshape(1, H),
    )
